# baseline (speedup 1.0000x reference)
import jax
import jax.numpy as jnp
from jax import lax
from jax.experimental import pallas as pl
from jax.experimental.pallas import tpu as pltpu
from jax.experimental.pallas import tpu_sc as plsc

B = 1024
L = 50
EMB = 128
HID = 512
NCLASS = 100000


_NC = 2
_NS = 16
_NW = _NC * _NS
_RPW = B // _NW
_LANES = 16
_CHUNKS = EMB // _LANES


def _sc_pool_body(seqs_hbm, table_hbm, out_hbm, idx_v, rows_v, acc_v, sem):
    wid = lax.axis_index("s") * _NC + lax.axis_index("c")
    base = wid * _RPW
    pltpu.sync_copy(seqs_hbm.at[pl.ds(base, _RPW)], idx_v)

    def per_row(i, carry):
        pltpu.async_copy(table_hbm.at[idx_v.at[i]], rows_v, sem).wait()

        def red(j, acc):
            return tuple(acc[c] + rows_v[j, pl.ds(c * _LANES, _LANES)]
                         for c in range(_CHUNKS))

        acc = lax.fori_loop(
            0, L, red,
            tuple(jnp.zeros((_LANES,), jnp.float32) for _ in range(_CHUNKS)))
        for c in range(_CHUNKS):
            acc_v[i, pl.ds(c * _LANES, _LANES)] = acc[c]
        return carry

    lax.fori_loop(0, _RPW, per_row, 0)
    pltpu.sync_copy(acc_v, out_hbm.at[pl.ds(base, _RPW)])


def _sc_pool(seqs, table):
    mesh = plsc.VectorSubcoreMesh(core_axis_name="c", subcore_axis_name="s")
    return pl.kernel(
        _sc_pool_body,
        out_type=jax.ShapeDtypeStruct((B, EMB), jnp.float32),
        mesh=mesh,
        scratch_types=[
            pltpu.VMEM((_RPW, L), jnp.int32),
            pltpu.VMEM((L, EMB), jnp.float32),
            pltpu.VMEM((_RPW, EMB), jnp.float32),
            pltpu.SemaphoreType.DMA,
        ],
    )(seqs, table)


TILE_N = 2048
NT = (NCLASS + TILE_N - 1) // TILE_N
_NEG = -1e30


def _h_body(s_ref, w1_ref, b1_ref, ht_ref):
    h = jnp.dot(s_ref[...], w1_ref[...],
                preferred_element_type=jnp.float32) + b1_ref[...]
    ht_ref[...] = jnp.transpose(jnp.maximum(h, 0.0)).astype(jnp.bfloat16)


def _lse_step(t, ht_ref, wt_ref, b2_ref, lz_ref, sc16_ref, m_sc, s_sc):
    @pl.when(t == 0)
    def _init():
        m_sc[...] = jnp.full_like(m_sc, _NEG)
        s_sc[...] = jnp.zeros_like(s_sc)

    row = t * TILE_N + lax.broadcasted_iota(jnp.int32, (TILE_N, 1), 0)
    valid = row < NCLASS
    w = jnp.where(valid, wt_ref[...], 0.0).astype(jnp.bfloat16)
    b2c = jnp.where(valid, b2_ref[...], _NEG)
    sc_t = jnp.dot(w, ht_ref[...], preferred_element_type=jnp.float32) + b2c
    sc16_ref[...] = sc_t.astype(jnp.bfloat16)
    m_old = m_sc[...]
    m_new = jnp.maximum(m_old, jnp.max(sc_t, axis=0, keepdims=True))
    s_sc[...] = (s_sc[...] * jnp.exp(m_old - m_new)
                 + jnp.sum(jnp.exp(sc_t - m_new), axis=0, keepdims=True))
    m_sc[...] = m_new

    @pl.when(t == NT - 1)
    def _fin():
        lz_ref[...] = m_sc[...] + jnp.log(s_sc[...])


def _lse_body(ht_ref, wt_ref, b2_ref, lz_ref, sc16_ref, m_sc, s_sc):
    _lse_step(pl.program_id(0), ht_ref, wt_ref, b2_ref, lz_ref, sc16_ref,
              m_sc, s_sc)


def _mid_body(ht_ref, wt_ref, b2_ref, scA_ref, lzA_ref,
              lz_ref, sc16_ref, out_ref, m_sc, s_sc):
    out_ref[...] = scA_ref[...].astype(jnp.float32) - lzA_ref[...]
    _lse_step(pl.program_id(0), ht_ref, wt_ref, b2_ref, lz_ref, sc16_ref,
              m_sc, s_sc)


def _fix_body(sc_ref, lz_ref, full_ref, o_ref):
    del full_ref
    o_ref[...] = sc_ref[...].astype(jnp.float32) - lz_ref[...]


def kernel(seqs, table, W1, b1, W2, b2):
    seqs = seqs.astype(jnp.int32)
    b1r = b1.reshape(1, HID)
    wt = jnp.transpose(W2)
    b2c = b2.reshape(NCLASS, 1)

    summed = _sc_pool(seqs, table)

    ht = pl.pallas_call(
        _h_body,
        out_shape=jax.ShapeDtypeStruct((HID, B), jnp.bfloat16),
    )(summed, W1, b1r)

    BH = B // 2

    lzA, scA = pl.pallas_call(
        _lse_body,
        grid=(NT,),
        in_specs=[
            pl.BlockSpec((HID, BH), lambda t: (0, 0)),
            pl.BlockSpec((TILE_N, HID), lambda t: (t, 0)),
            pl.BlockSpec((TILE_N, 1), lambda t: (t, 0)),
        ],
        out_specs=[
            pl.BlockSpec((1, BH), lambda t: (0, 0)),
            pl.BlockSpec((TILE_N, BH), lambda t: (t, 0)),
        ],
        out_shape=[
            jax.ShapeDtypeStruct((1, BH), jnp.float32),
            jax.ShapeDtypeStruct((NCLASS, BH), jnp.bfloat16),
        ],
        scratch_shapes=[
            pltpu.VMEM((1, BH), jnp.float32),
            pltpu.VMEM((1, BH), jnp.float32),
        ],
        compiler_params=pltpu.CompilerParams(
            dimension_semantics=("arbitrary",)),
    )(ht, wt, b2c)

    lzB, scB, out_t = pl.pallas_call(
        _mid_body,
        grid=(NT,),
        in_specs=[
            pl.BlockSpec((HID, BH), lambda t: (0, 1)),
            pl.BlockSpec((TILE_N, HID), lambda t: (t, 0)),
            pl.BlockSpec((TILE_N, 1), lambda t: (t, 0)),
            pl.BlockSpec((TILE_N, BH), lambda t: (t, 0)),
            pl.BlockSpec((1, BH), lambda t: (0, 0)),
        ],
        out_specs=[
            pl.BlockSpec((1, BH), lambda t: (0, 0)),
            pl.BlockSpec((TILE_N, BH), lambda t: (t, 0)),
            pl.BlockSpec((TILE_N, BH), lambda t: (t, 0)),
        ],
        out_shape=[
            jax.ShapeDtypeStruct((1, BH), jnp.float32),
            jax.ShapeDtypeStruct((NCLASS, BH), jnp.bfloat16),
            jax.ShapeDtypeStruct((NCLASS, B), jnp.float32),
        ],
        scratch_shapes=[
            pltpu.VMEM((1, BH), jnp.float32),
            pltpu.VMEM((1, BH), jnp.float32),
        ],
        compiler_params=pltpu.CompilerParams(
            dimension_semantics=("arbitrary",)),
    )(ht, wt, b2c, scA, lzA)

    out_t = pl.pallas_call(
        _fix_body,
        grid=(NT,),
        in_specs=[
            pl.BlockSpec((TILE_N, BH), lambda t: (t, 0)),
            pl.BlockSpec((1, BH), lambda t: (0, 0)),
            pl.BlockSpec(memory_space=pltpu.MemorySpace.HBM),
        ],
        out_specs=pl.BlockSpec((TILE_N, BH), lambda t: (t, 1)),
        out_shape=jax.ShapeDtypeStruct((NCLASS, B), jnp.float32),
        input_output_aliases={2: 0},
        compiler_params=pltpu.CompilerParams(
            dimension_semantics=("parallel",)),
    )(scB, lzB, out_t)

    return jnp.transpose(out_t)


# speedup vs baseline: 1.1441x; 1.1441x over previous
import jax
import jax.numpy as jnp
from jax import lax
from jax.experimental import pallas as pl
from jax.experimental.pallas import tpu as pltpu
from jax.experimental.pallas import tpu_sc as plsc

B = 1024
L = 50
EMB = 128
HID = 512
NCLASS = 100000


_NC = 2
_NS = 16
_NW = _NC * _NS
_RPW = B // _NW
_LANES = 16
_CHUNKS = EMB // _LANES


def _sc_pool_body(seqs_hbm, table_hbm, out_hbm, idx_v, rows_v, acc_v, sem):
    wid = lax.axis_index("s") * _NC + lax.axis_index("c")
    base = wid * _RPW
    pltpu.sync_copy(seqs_hbm.at[pl.ds(base, _RPW)], idx_v)

    def per_row(i, carry):
        pltpu.async_copy(table_hbm.at[idx_v.at[i]], rows_v, sem).wait()

        def red(j, acc):
            return tuple(acc[c] + rows_v[j, pl.ds(c * _LANES, _LANES)]
                         for c in range(_CHUNKS))

        acc = lax.fori_loop(
            0, L, red,
            tuple(jnp.zeros((_LANES,), jnp.float32) for _ in range(_CHUNKS)))
        for c in range(_CHUNKS):
            acc_v[i, pl.ds(c * _LANES, _LANES)] = acc[c]
        return carry

    lax.fori_loop(0, _RPW, per_row, 0)
    pltpu.sync_copy(acc_v, out_hbm.at[pl.ds(base, _RPW)])


def _sc_pool(seqs, table):
    mesh = plsc.VectorSubcoreMesh(core_axis_name="c", subcore_axis_name="s")
    return pl.kernel(
        _sc_pool_body,
        out_type=jax.ShapeDtypeStruct((B, EMB), jnp.float32),
        mesh=mesh,
        scratch_types=[
            pltpu.VMEM((_RPW, L), jnp.int32),
            pltpu.VMEM((L, EMB), jnp.float32),
            pltpu.VMEM((_RPW, EMB), jnp.float32),
            pltpu.SemaphoreType.DMA,
        ],
    )(seqs, table)


TILE_N = 4096
NT = (NCLASS + TILE_N - 1) // TILE_N
_NEG = -1e30


def _h_body(s_ref, w1_ref, b1_ref, ht_ref):
    h = jnp.dot(s_ref[...], w1_ref[...],
                preferred_element_type=jnp.float32) + b1_ref[...]
    ht_ref[...] = jnp.transpose(jnp.maximum(h, 0.0)).astype(jnp.bfloat16)


def _lse_body(ht_ref, wt_ref, b2_ref, lz_ref, sc16_ref, m_sc, s_sc):
    t = pl.program_id(0)

    @pl.when(t == 0)
    def _init():
        m_sc[...] = jnp.full_like(m_sc, _NEG)
        s_sc[...] = jnp.zeros_like(s_sc)

    row = t * TILE_N + lax.broadcasted_iota(jnp.int32, (TILE_N, 1), 0)
    valid = row < NCLASS
    w = jnp.where(valid, wt_ref[...], 0.0).astype(jnp.bfloat16)
    b2c = jnp.where(valid, b2_ref[...], _NEG)
    sc_t = jnp.dot(w, ht_ref[...], preferred_element_type=jnp.float32) + b2c
    sc16_ref[...] = sc_t.astype(jnp.bfloat16)
    m_old = m_sc[...]
    m_new = jnp.maximum(m_old, jnp.max(sc_t, axis=0, keepdims=True))
    s_sc[...] = (s_sc[...] * jnp.exp(m_old - m_new)
                 + jnp.sum(jnp.exp(sc_t - m_new), axis=0, keepdims=True))
    m_sc[...] = m_new

    @pl.when(t == NT - 1)
    def _fin():
        lz_ref[...] = m_sc[...] + jnp.log(s_sc[...])


def _fix_body(sc_ref, lz_ref, o_ref):
    o_ref[...] = sc_ref[...].astype(jnp.float32) - lz_ref[...]


def kernel(seqs, table, W1, b1, W2, b2):
    seqs = seqs.astype(jnp.int32)
    b1r = b1.reshape(1, HID)
    wt = jnp.transpose(W2)
    b2c = b2.reshape(NCLASS, 1)

    summed = _sc_pool(seqs, table)

    ht = pl.pallas_call(
        _h_body,
        out_shape=jax.ShapeDtypeStruct((HID, B), jnp.bfloat16),
    )(summed, W1, b1r)

    logZ, scores16 = pl.pallas_call(
        _lse_body,
        grid=(NT,),
        in_specs=[
            pl.BlockSpec((HID, B), lambda t: (0, 0)),
            pl.BlockSpec((TILE_N, HID), lambda t: (t, 0)),
            pl.BlockSpec((TILE_N, 1), lambda t: (t, 0)),
        ],
        out_specs=[
            pl.BlockSpec((1, B), lambda t: (0, 0)),
            pl.BlockSpec((TILE_N, B), lambda t: (t, 0)),
        ],
        out_shape=[
            jax.ShapeDtypeStruct((1, B), jnp.float32),
            jax.ShapeDtypeStruct((NCLASS, B), jnp.bfloat16),
        ],
        scratch_shapes=[
            pltpu.VMEM((1, B), jnp.float32),
            pltpu.VMEM((1, B), jnp.float32),
        ],
        compiler_params=pltpu.CompilerParams(
            dimension_semantics=("arbitrary",)),
    )(ht, wt, b2c)

    out_t = pl.pallas_call(
        _fix_body,
        grid=(NT,),
        in_specs=[
            pl.BlockSpec((TILE_N, B), lambda t: (t, 0)),
            pl.BlockSpec((1, B), lambda t: (0, 0)),
        ],
        out_specs=pl.BlockSpec((TILE_N, B), lambda t: (t, 0)),
        out_shape=jax.ShapeDtypeStruct((NCLASS, B), jnp.float32),
        compiler_params=pltpu.CompilerParams(
            dimension_semantics=("parallel",)),
    )(scores16, logZ)

    return jnp.transpose(out_t)
